# Initial kernel scaffold; baseline (speedup 1.0000x reference)
import jax
import jax.numpy as jnp
from jax.experimental import pallas as pl


def kernel(x, batch):
    raise NotImplementedError("write your pallas kernel here")


# speedup vs baseline: 20.6602x; 20.6602x over previous
import jax
import jax.numpy as jnp
from jax.experimental import pallas as pl
from jax.experimental.pallas import tpu as pltpu

NSEG = 128
ROWS = 50000
D = 512
R = 2000


def _body(b_ref, x_ref, o_ref, acc_ref, cnt_ref):
    i = pl.program_id(0)

    @pl.when(i == 0)
    def _init():
        acc_ref[...] = jnp.zeros_like(acc_ref)
        cnt_ref[...] = jnp.zeros_like(cnt_ref)

    b = b_ref[0, 0, :]
    seg = jax.lax.broadcasted_iota(jnp.int32, (NSEG, R), 0)
    onehot_t = (seg == b[None, :]).astype(jnp.float32)
    acc_ref[...] += jnp.dot(onehot_t, x_ref[...],
                            preferred_element_type=jnp.float32)
    cnt_ref[...] += jnp.sum(onehot_t, axis=1, keepdims=True)

    @pl.when(i == pl.num_programs(0) - 1)
    def _finish():
        o_ref[...] = acc_ref[...] / jnp.maximum(cnt_ref[...], 1.0)


def kernel(x, batch):
    nblk = ROWS // R
    b3 = batch.astype(jnp.int32).reshape(nblk, 1, R)
    return pl.pallas_call(
        _body,
        grid=(nblk,),
        in_specs=[
            pl.BlockSpec((1, 1, R), lambda i: (i, 0, 0)),
            pl.BlockSpec((R, D), lambda i: (i, 0)),
        ],
        out_specs=pl.BlockSpec((NSEG, D), lambda i: (0, 0)),
        out_shape=jax.ShapeDtypeStruct((NSEG, D), jnp.float32),
        scratch_shapes=[
            pltpu.VMEM((NSEG, D), jnp.float32),
            pltpu.VMEM((NSEG, 1), jnp.float32),
        ],
    )(b3, x)
